# baseline (speedup 1.0000x reference)
import functools
import math

import jax
import jax.numpy as jnp
from jax import lax
from jax.experimental import pallas as pl
from jax.experimental.pallas import tpu as pltpu
from jax.experimental.pallas import tpu_sc as plsc

SEQ = 200
D = 64
BATCH = 4096
N = BATCH * SEQ

NC = 2
NS = 16
NW = NC * NS

NBB = BATCH // 128
UNITS = NBB // 2
SCALE = math.sqrt(float(D))

HSPLIT = 500224
TC_BLK = 512
TC_GRID = HSPLIT // TC_BLK


def _positional_encoding():
    position = jnp.arange(0, SEQ, 1, dtype=jnp.float32).reshape(-1, 1)
    multiplication = jnp.exp(
        -jnp.arange(0, D * 2, 2, dtype=jnp.float32) * math.log(10000.0) / D)
    excessive = position * multiplication
    pe = jnp.zeros((SEQ, D), dtype=jnp.float32)
    pe = pe.at[:, 0::2].set(jnp.sin(excessive[:, 0::2]))
    pe = pe.at[:, 1::2].set(jnp.cos(excessive[:, 1::2]))
    return pe


def _body(t_hbm, s_hbm, pe_hbm, out_hbm, idx_v, pe_v, rows0, rows1, stage0,
          stage1, gsem0, gsem1, osem0, osem1):
    wid = lax.axis_index("s") * NC + lax.axis_index("c")
    iota = lax.iota(jnp.int32, 16)
    rows = (rows0, rows1)
    stage = (stage0, stage1)
    gsem = (gsem0, gsem1)
    osem = (osem0, osem1)

    def issue_gather(u, b):
        pltpu.async_copy(t_hbm.at[idx_v.at[2 * u]],
                         rows[b].at[pl.ds(0, 128)], gsem[b])
        pltpu.async_copy(t_hbm.at[idx_v.at[2 * u + 1]],
                         rows[b].at[pl.ds(128, 128)], gsem[b])

    def wait_gather(u, b):
        pltpu.make_async_copy(t_hbm.at[idx_v.at[2 * u]],
                              rows[b].at[pl.ds(0, 128)], gsem[b]).wait()
        pltpu.make_async_copy(t_hbm.at[idx_v.at[2 * u + 1]],
                              rows[b].at[pl.ds(128, 128)], gsem[b]).wait()

    def wait_out(s, u, b):
        pltpu.make_async_copy(stage[b], out_hbm.at[s, :, pl.ds(2 * u, 2)],
                              osem[b]).wait()

    def slab_loop(si, carry):
        s = si * NW + wid
        pltpu.sync_copy(s_hbm.at[s], idx_v)
        pltpu.sync_copy(pe_hbm.at[s], pe_v)

        @functools.partial(plsc.parallel_loop, 0, NBB, unroll=4)
        def remap_loop(r):
            for c in range(8):
                v = idx_v[r, pl.ds(c * 16, 16)]
                idx_v[r, pl.ds(c * 16, 16)] = 2 * v - jnp.where(
                    v < HSPLIT, 0, 2 * HSPLIT - 1)

        issue_gather(0, 0)
        issue_gather(1, 1)

        def pair_loop(p, c2):
            for b in range(2):
                u = 2 * p + b
                wait_gather(u, b)
                @pl.when(p > 0)
                def _():
                    wait_out(s, u - 2, b)

                rb = rows[b]
                sb = stage[b]

                @functools.partial(plsc.parallel_loop, 0, D, unroll=8)
                def d_loop(d):
                    dblk = d // 8
                    dr = d - dblk * 8
                    pe_vec = pe_v[pl.ds(d * 16, 16)]
                    col_idx = jnp.broadcast_to(d, (16,)).astype(jnp.int32)
                    for j in range(16):
                        v = plsc.load_gather(rb, [iota + (j * 16), col_idx])
                        sb[dblk, j // 8, dr, pl.ds((j % 8) * 16, 16)] = (
                            v * SCALE + pe_vec)

                pltpu.async_copy(sb, out_hbm.at[s, :, pl.ds(2 * u, 2)],
                                 osem[b])

                @pl.when(p < UNITS // 2 - 1)
                def _():
                    issue_gather(u + 2, b)
            return c2

        lax.fori_loop(0, UNITS // 2, pair_loop, 0)
        wait_out(s, UNITS - 2, 0)
        wait_out(s, UNITS - 1, 1)
        return carry

    nslab = jnp.where(wid < (SEQ - (SEQ // NW) * NW), SEQ // NW + 1, SEQ // NW)
    lax.fori_loop(0, nslab, slab_loop, 0)


def _tc_pack(tt):
    def body(a_ref, b_ref, o_ref):
        eye = jnp.eye(D, dtype=jnp.float32)
        dn = (((0,), (0,)), ((), ()))
        ta = lax.dot_general(a_ref[...], eye, dn,
                             preferred_element_type=jnp.float32)
        tb = lax.dot_general(b_ref[...], eye, dn,
                             preferred_element_type=jnp.float32)
        o_ref[...] = jnp.concatenate([ta, tb], axis=1)

    return pl.pallas_call(
        body,
        grid=(TC_GRID,),
        in_specs=[
            pl.BlockSpec((D, TC_BLK), lambda k: (0, k)),
            pl.BlockSpec((D, TC_BLK), lambda k: (0, TC_GRID + k)),
        ],
        out_specs=pl.BlockSpec((TC_BLK, 128), lambda k: (k, 0)),
        out_shape=jax.ShapeDtypeStruct((HSPLIT, 128), jnp.float32),
    )(tt, tt)


@jax.jit
def _run(stream_t3, table, pe_rep):
    mesh = plsc.VectorSubcoreMesh(core_axis_name="c", subcore_axis_name="s",
                                  num_cores=NC, num_subcores=NS)
    f = pl.kernel(
        _body,
        out_type=jax.ShapeDtypeStruct((SEQ, 8, NBB, 8, 128), jnp.float32),
        mesh=mesh,
        scratch_types=[
            pltpu.VMEM((NBB, 128), jnp.int32),
            pltpu.VMEM((D * 16,), jnp.float32),
            pltpu.VMEM((256, D), jnp.float32),
            pltpu.VMEM((256, D), jnp.float32),
            pltpu.VMEM((8, 2, 8, 128), jnp.float32),
            pltpu.VMEM((8, 2, 8, 128), jnp.float32),
            pltpu.SemaphoreType.DMA,
            pltpu.SemaphoreType.DMA,
            pltpu.SemaphoreType.DMA,
            pltpu.SemaphoreType.DMA,
        ],
        compiler_params=pltpu.CompilerParams(use_tc_tiling_on_sc=False,
                                             needs_layout_passes=False),
    )
    return f(table, stream_t3, pe_rep)


def kernel(stream, table):
    pe = _positional_encoding()
    pe_rep = jnp.repeat(pe, 16, axis=1)
    stream_t3 = stream.T.reshape(SEQ, NBB, 128)
    t2 = _tc_pack(table.T)
    t2v = t2.reshape(2 * HSPLIT, D)
    out5 = _run(stream_t3, t2v, pe_rep)
    return out5.transpose(2, 4, 0, 1, 3).reshape(BATCH, SEQ, D)


# speedup vs baseline: 1.1664x; 1.1664x over previous
import functools
import math

import jax
import jax.numpy as jnp
from jax import lax
from jax.experimental import pallas as pl
from jax.experimental.pallas import tpu as pltpu
from jax.experimental.pallas import tpu_sc as plsc

SEQ = 200
D = 64
BATCH = 4096
N = BATCH * SEQ

NC = 2
NS = 16
NW = NC * NS

NBB = BATCH // 128
UNITS = NBB // 2
SCALE = math.sqrt(float(D))


def _positional_encoding():
    position = jnp.arange(0, SEQ, 1, dtype=jnp.float32).reshape(-1, 1)
    multiplication = jnp.exp(
        -jnp.arange(0, D * 2, 2, dtype=jnp.float32) * math.log(10000.0) / D)
    excessive = position * multiplication
    pe = jnp.zeros((SEQ, D), dtype=jnp.float32)
    pe = pe.at[:, 0::2].set(jnp.sin(excessive[:, 0::2]))
    pe = pe.at[:, 1::2].set(jnp.cos(excessive[:, 1::2]))
    return pe


def _body(t_hbm, s_hbm, pe_hbm, out_hbm, idx_v, pe_v, rows0, rows1, stage0,
          stage1, gsem0, gsem1, osem0, osem1):
    wid = lax.axis_index("s") * NC + lax.axis_index("c")
    iota = lax.iota(jnp.int32, 16)
    rows = (rows0, rows1)
    stage = (stage0, stage1)
    gsem = (gsem0, gsem1)
    osem = (osem0, osem1)

    def issue_gather(u, b):
        pltpu.async_copy(t_hbm.at[idx_v.at[2 * u]],
                         rows[b].at[pl.ds(0, 128)], gsem[b])
        pltpu.async_copy(t_hbm.at[idx_v.at[2 * u + 1]],
                         rows[b].at[pl.ds(128, 128)], gsem[b])

    def wait_gather(u, b):
        pltpu.make_async_copy(t_hbm.at[idx_v.at[2 * u]],
                              rows[b].at[pl.ds(0, 128)], gsem[b]).wait()
        pltpu.make_async_copy(t_hbm.at[idx_v.at[2 * u + 1]],
                              rows[b].at[pl.ds(128, 128)], gsem[b]).wait()

    def wait_out(s, u, b):
        pltpu.make_async_copy(stage[b], out_hbm.at[s, :, pl.ds(2 * u, 2)],
                              osem[b]).wait()

    def slab_loop(si, carry):
        s = si * NW + wid
        pltpu.sync_copy(s_hbm.at[s], idx_v)
        pltpu.sync_copy(pe_hbm.at[s], pe_v)
        issue_gather(0, 0)
        issue_gather(1, 1)

        def pair_loop(p, c2):
            for b in range(2):
                u = 2 * p + b
                wait_gather(u, b)
                @pl.when(p > 0)
                def _():
                    wait_out(s, u - 2, b)

                rb = rows[b]
                sb = stage[b]

                @functools.partial(plsc.parallel_loop, 0, D, unroll=8)
                def d_loop(d):
                    dblk = d // 8
                    dr = d - dblk * 8
                    pe_vec = pe_v[pl.ds(d * 16, 16)]
                    col_idx = jnp.broadcast_to(d, (16,)).astype(jnp.int32)
                    for j in range(16):
                        v = plsc.load_gather(rb, [iota + (j * 16), col_idx])
                        sb[dblk, j // 8, dr, pl.ds((j % 8) * 16, 16)] = (
                            v * SCALE + pe_vec)

                pltpu.async_copy(sb, out_hbm.at[s, :, pl.ds(2 * u, 2)],
                                 osem[b])

                @pl.when(p < UNITS // 2 - 1)
                def _():
                    issue_gather(u + 2, b)
            return c2

        lax.fori_loop(0, UNITS // 2, pair_loop, 0)
        wait_out(s, UNITS - 2, 0)
        wait_out(s, UNITS - 1, 1)
        return carry

    nslab = jnp.where(wid < (SEQ - (SEQ // NW) * NW), SEQ // NW + 1, SEQ // NW)
    lax.fori_loop(0, nslab, slab_loop, 0)


@jax.jit
def _run(stream_t3, table, pe_rep):
    mesh = plsc.VectorSubcoreMesh(core_axis_name="c", subcore_axis_name="s",
                                  num_cores=NC, num_subcores=NS)
    f = pl.kernel(
        _body,
        out_type=jax.ShapeDtypeStruct((SEQ, 8, NBB, 8, 128), jnp.float32),
        mesh=mesh,
        scratch_types=[
            pltpu.VMEM((NBB, 128), jnp.int32),
            pltpu.VMEM((D * 16,), jnp.float32),
            pltpu.VMEM((256, D), jnp.float32),
            pltpu.VMEM((256, D), jnp.float32),
            pltpu.VMEM((8, 2, 8, 128), jnp.float32),
            pltpu.VMEM((8, 2, 8, 128), jnp.float32),
            pltpu.SemaphoreType.DMA,
            pltpu.SemaphoreType.DMA,
            pltpu.SemaphoreType.DMA,
            pltpu.SemaphoreType.DMA,
        ],
        compiler_params=pltpu.CompilerParams(use_tc_tiling_on_sc=False,
                                             needs_layout_passes=False),
    )
    return f(table, stream_t3, pe_rep)


def kernel(stream, table):
    pe = _positional_encoding()
    pe_rep = jnp.repeat(pe, 16, axis=1)
    stream_t3 = stream.T.reshape(SEQ, NBB, 128)
    out5 = _run(stream_t3, table, pe_rep)
    return out5.transpose(2, 4, 0, 1, 3).reshape(BATCH, SEQ, D)
